# baseline (speedup 1.0000x reference)
import functools

import jax
import jax.numpy as jnp
from jax import lax
from jax.experimental import pallas as pl
from jax.experimental.pallas import tpu as pltpu
from jax.experimental.pallas import tpu_sc as plsc

_RADIUS2 = 0.25
_S = 32
_PBLK = 256
_NCHUNK = 2048
_HPW = 16
_WPW = 32
_NTILES = 32
_RING = 2


def _bq_body(new_ref, xyzt_ref, words_ref):
    newb = new_ref[0]
    xyzt = xyzt_ref[0]
    n_pts = xyzt.shape[1]
    nchunks = n_pts // _NCHUNK
    nwc = _NCHUNK // _WPW

    sq_c = jnp.sum(newb * newb, axis=1, keepdims=True)

    nn = lax.broadcasted_iota(jnp.int32, (_NCHUNK, 2 * nwc), 0)
    cc = lax.broadcasted_iota(jnp.int32, (_NCHUNK, 2 * nwc), 1)
    is_lo = cc < nwc
    wsel = jnp.where(is_lo, cc, cc - nwc)
    in_word = (nn // _WPW) == wsel
    bitpos = nn % _WPW
    in_half = jnp.logical_xor(is_lo, bitpos >= _HPW)
    pw = (1 << (bitpos % _HPW)).astype(jnp.float32)
    wmat = jnp.where(in_word, jnp.where(in_half, pw, 0.0),
                     0.0).astype(jnp.bfloat16)

    lo_cols = []
    hi_cols = []
    for ci in range(nchunks):
        xc = lax.slice(xyzt, (0, ci * _NCHUNK), (3, (ci + 1) * _NCHUNK))
        cross = jnp.dot(newb, xc, preferred_element_type=jnp.float32)
        sq_x = jnp.sum(xc * xc, axis=0, keepdims=True)
        dist2 = sq_c + sq_x - 2.0 * cross
        m = (dist2 < _RADIUS2).astype(jnp.bfloat16)
        wc = jnp.dot(m, wmat, preferred_element_type=jnp.float32)
        lo_cols.append(lax.slice(wc, (0, 0), (_PBLK, nwc)))
        hi_cols.append(lax.slice(wc, (0, nwc), (_PBLK, 2 * nwc)))
    lo = jnp.concatenate(lo_cols, axis=1).astype(jnp.int32)
    hi = jnp.concatenate(hi_cols, axis=1).astype(jnp.int32)
    words = jnp.bitwise_or(lo, hi << _HPW)
    words_ref[0] = jnp.transpose(words, (1, 0))


def _ball_query_words(new_xyz, xyz_t):
    b, p, _ = new_xyz.shape
    n = xyz_t.shape[2]
    nw = n // _WPW
    return pl.pallas_call(
        _bq_body,
        grid=(b, p // _PBLK),
        in_specs=[
            pl.BlockSpec((1, _PBLK, 3), lambda bi, ji: (bi, ji, 0)),
            pl.BlockSpec((1, 3, n), lambda bi, ji: (bi, 0, 0)),
        ],
        out_specs=pl.BlockSpec((1, nw, _PBLK), lambda bi, ji: (bi, 0, ji)),
        out_shape=jax.ShapeDtypeStruct((b, nw, p), jnp.int32),
    )(new_xyz, xyz_t)


def _sc_select_gather(wordst, feats, xyzf, nxpf, b_sz, c_sz, n, p):
    ntb = _NTILES // b_sz
    pchunk = p // ntb
    nvp = pchunk // 16
    nw = n // _WPW
    nwh = nw // 2
    lgs = pchunk // 16
    out_ch = c_sz + 3

    mesh = plsc.VectorSubcoreMesh(core_axis_name="c", subcore_axis_name="s")

    def body(words_hbm, feats_hbm, xyzf_hbm, nxp_hbm, out_hbm,
             wv_v, gi_v, mz_v, cnt_v, xyzrow_v, nxp_v,
             row0, row1, oxyz, o0, o1, dump_sh,
             rs0, rs1, os0, os1):
        rows = [row0, row1]
        outs = [o0, o1]
        rowsems = [rs0, rs1]
        outsems = [os0, os1]
        cid = lax.axis_index("c")
        sid = lax.axis_index("s")
        wid = sid * 2 + cid
        bb = wid // ntb
        pc = wid % ntb
        p0 = pc * pchunk
        pltpu.sync_copy(xyzf_hbm.at[pl.ds(bb * 3 * n, 3 * n)],
                        xyzrow_v.at[pl.ds(0, 3 * n)])
        xyzrow_v[pl.ds(3 * n, 16)] = jnp.zeros((16,), jnp.float32)
        for d in range(3):
            pltpu.sync_copy(
                nxp_hbm.at[pl.ds((bb * 3 + d) * p + p0, pchunk)],
                nxp_v.at[pl.ds(d * pchunk, pchunk)])

        @plsc.parallel_loop(0, _S * nvp, 1, unroll=4)
        def _(v):
            s = v >> 4
            off = (v & (nvp - 1)) * 16
            gi_v[s, pl.ds(off, 16)] = jnp.full((16,), n, jnp.int32)
            mz_v[s, pl.ds(off, 16)] = jnp.zeros((16,), jnp.float32)

        @plsc.parallel_loop(0, nvp, 1, unroll=4)
        def _(v):
            cnt_v[pl.ds(v * 16, 16)] = jnp.zeros((16,), jnp.int32)

        lanei = lax.broadcasted_iota(jnp.int32, (16,), 0)
        ones16 = jnp.ones((16,), jnp.float32)
        for half in range(2):
            pltpu.sync_copy(
                words_hbm.at[bb, pl.ds(half * nwh, nwh),
                             pl.ds(p0, pchunk)], wv_v)
            for lg in range(lgs):
                cols = lg * 16 + lanei

                def wloop(w, cnt, lg=lg, cols=cols):
                    v32 = wv_v[w, pl.ds(lg * 16, 16)]
                    vu = plsc.bitcast(v32, jnp.uint32)

                    def cond(st):
                        vv, cc2 = st
                        act = jnp.logical_and(vv != 0, cc2 < _S)
                        return lax.reduce_or(act, axes=(0,))

                    def step(st, half=half):
                        vv, cc2 = st
                        act = jnp.logical_and(vv != 0, cc2 < _S)
                        low = vv & (~vv + 1)
                        pos = (plsc.bitcast(low.astype(jnp.float32),
                                            jnp.int32) >> 23) - 127
                        wglob = (half * nwh + w) * _WPW
                        n_idx = wglob + pos
                        slot = jnp.minimum(cc2, _S - 1)
                        plsc.store_scatter(gi_v, [slot, cols], n_idx,
                                           mask=act)
                        plsc.store_scatter(mz_v, [slot, cols], ones16,
                                           mask=act)
                        vv = jnp.where(act, vv & (vv - 1), vv)
                        cc2 = jnp.where(act, cc2 + 1, cc2)
                        return (vv, cc2)

                    vu, cnt = lax.while_loop(cond, step, (vu, cnt))
                    return cnt

                cnt0 = cnt_v[pl.ds(lg * 16, 16)]
                cnt1 = lax.fori_loop(0, nwh, wloop, cnt0)
                cnt_v[pl.ds(lg * 16, 16)] = cnt1

        for d in range(3):
            @plsc.parallel_loop(0, _S * nvp, 1, unroll=4)
            def _(v, d=d):
                s = v >> 4
                off = (v & (nvp - 1)) * 16
                g = gi_v[s, pl.ds(off, 16)]
                vals = plsc.load_gather(xyzrow_v, [g * 3 + d])
                nx = nxp_v[pl.ds(d * pchunk + off, 16)]
                m = mz_v[s, pl.ds(off, 16)]
                oxyz[s, pl.ds(off, 16)] = (vals - nx) * m
            pltpu.sync_copy(oxyz, out_hbm.at[bb, d, :, pl.ds(p0, pchunk)])

        for r in range(_RING):
            pltpu.async_copy(
                feats_hbm.at[pl.ds((bb * c_sz + r) * n, n)],
                rows[r].at[pl.ds(0, n)], rowsems[r])
            rows[r][pl.ds(n, 16)] = jnp.zeros((16,), jnp.float32)
            pltpu.async_copy(outs[r], dump_sh, outsems[r])

        def chan_group(k, carry):
            for r in range(_RING):
                c = k * _RING + r
                pltpu.make_async_copy(
                    feats_hbm.at[pl.ds((bb * c_sz) * n, n)],
                    rows[r].at[pl.ds(0, n)], rowsems[r]).wait()
                pltpu.make_async_copy(outs[r], dump_sh, outsems[r]).wait()

                @plsc.parallel_loop(0, _S * nvp, 1, unroll=8)
                def _(v, r=r):
                    s = v >> 4
                    off = (v & (nvp - 1)) * 16
                    g = gi_v[s, pl.ds(off, 16)]
                    vals = plsc.load_gather(rows[r], [g])
                    outs[r][s, pl.ds(off, 16)] = vals

                pltpu.async_copy(
                    outs[r],
                    out_hbm.at[bb, 3 + c, :, pl.ds(p0, pchunk)],
                    outsems[r])
                cnext = jnp.minimum(c + _RING, c_sz - 1)
                pltpu.async_copy(
                    feats_hbm.at[pl.ds((bb * c_sz + cnext) * n, n)],
                    rows[r].at[pl.ds(0, n)], rowsems[r])
            return carry
        lax.fori_loop(0, c_sz // _RING, chan_group, 0)

        for r in range(_RING):
            pltpu.make_async_copy(
                feats_hbm.at[pl.ds((bb * c_sz) * n, n)],
                rows[r].at[pl.ds(0, n)], rowsems[r]).wait()
            pltpu.make_async_copy(outs[r], dump_sh, outsems[r]).wait()

    k = pl.kernel(
        body,
        out_type=jax.ShapeDtypeStruct((b_sz, out_ch, _S, p), jnp.float32),
        mesh=mesh,
        compiler_params=pltpu.CompilerParams(needs_layout_passes=False),
        scratch_types=(
            [
                pltpu.VMEM((nwh, pchunk), jnp.int32),
                pltpu.VMEM((_S, pchunk), jnp.int32),
                pltpu.VMEM((_S, pchunk), jnp.float32),
                pltpu.VMEM((pchunk,), jnp.int32),
                pltpu.VMEM((3 * n + 16,), jnp.float32),
                pltpu.VMEM((3 * pchunk,), jnp.float32),
            ]
            + [pltpu.VMEM((n + 16,), jnp.float32)] * _RING
            + [pltpu.VMEM((_S, pchunk), jnp.float32)] * (1 + _RING)
            + [pltpu.VMEM_SHARED((_S, pchunk), jnp.float32)]
            + [pltpu.SemaphoreType.DMA] * (2 * _RING)
        ),
    )
    return k(wordst, feats, xyzf, nxpf)


def kernel(xyz, new_xyz, features):
    b, c, n = features.shape
    p = new_xyz.shape[1]
    xyz_t = jnp.transpose(xyz, (0, 2, 1))
    wordst = _ball_query_words(new_xyz, xyz_t)
    xyzf = xyz.reshape(b * 3 * n)
    nxpf = jnp.transpose(new_xyz, (0, 2, 1)).reshape(b * 3 * p)
    out = _sc_select_gather(wordst, features.reshape(b * c * n), xyzf,
                            nxpf, b, c, n, p)
    return jnp.transpose(out, (0, 1, 3, 2))


# speedup vs baseline: 1.2586x; 1.2586x over previous
import functools

import jax
import jax.numpy as jnp
from jax import lax
from jax.experimental import pallas as pl
from jax.experimental.pallas import tpu as pltpu
from jax.experimental.pallas import tpu_sc as plsc

_RADIUS2 = 0.25
_S = 32
_PBLK = 256
_NCHUNK = 2048
_WPW = 16
_NTILES = 32
_RING = 4


def _bq_body(new_ref, xyzt_ref, wmat_ref, tri_ref, wi16k_ref, idx_ref):
    newb = new_ref[0]
    xyzt = xyzt_ref[0]
    wmat = wmat_ref[...]
    tri = tri_ref[...]
    wi16k = wi16k_ref[...]
    n_pts = xyzt.shape[1]
    nchunks = n_pts // _NCHUNK
    nwc = _NCHUNK // _WPW
    nw = n_pts // _WPW

    sq_c = jnp.sum(newb * newb, axis=1, keepdims=True)

    words_cols = []
    cnts_cols = []
    for ci in range(nchunks):
        xc = lax.slice(xyzt, (0, ci * _NCHUNK), (3, (ci + 1) * _NCHUNK))
        cross = jnp.dot(newb, xc, preferred_element_type=jnp.float32)
        sq_x = jnp.sum(xc * xc, axis=0, keepdims=True)
        dist2 = sq_c + sq_x - 2.0 * cross
        m = (dist2 < _RADIUS2).astype(jnp.bfloat16)
        wc = jnp.dot(m, wmat, preferred_element_type=jnp.float32)
        words_cols.append(lax.slice(wc, (0, 0), (_PBLK, nwc)))
        cnts_cols.append(lax.slice(wc, (0, nwc), (_PBLK, 2 * nwc)))
    words = jnp.concatenate(words_cols, axis=1)
    cnts = jnp.concatenate(cnts_cols, axis=1)

    cum = jnp.dot(cnts.astype(jnp.bfloat16), tri,
                  preferred_element_type=jnp.float32)
    prev = cum - cnts
    total = lax.slice(cum, (0, nw - 1), (_PBLK, nw))

    combo_base = wi16k + prev
    combo_cols = []
    word_cols = []
    for s in range(_S):
        sf = float(s)
        mf = jnp.logical_and(prev <= sf, cum > sf).astype(jnp.float32)
        combo_cols.append(jnp.sum(mf * combo_base, axis=1, keepdims=True))
        word_cols.append(jnp.sum(mf * words, axis=1, keepdims=True))
    combo = jnp.concatenate(combo_cols, axis=1)
    wordv = jnp.concatenate(word_cols, axis=1)

    cwi = combo.astype(jnp.int32)
    w_i = cwi >> 14
    base = cwi & 16383
    wv = wordv.astype(jnp.int32)
    si = lax.broadcasted_iota(jnp.int32, (_PBLK, _S), 1)
    j = si - base
    run = jnp.zeros((_PBLK, _S), jnp.int32)
    pos = jnp.zeros((_PBLK, _S), jnp.int32)
    for k in range(_WPW):
        bit = (wv >> k) & 1
        if k > 0:
            hit = jnp.logical_and(bit > 0, run == j)
            pos = pos + jnp.where(hit, k, 0)
        run = run + bit
    n_idx = w_i * _WPW + pos
    valid = si < total.astype(jnp.int32)
    idx = jnp.where(valid, n_idx + 1, 0)
    idx_ref[0] = jnp.transpose(idx, (1, 0))


def _ball_query(new_xyz, xyz_t):
    b, p, _ = new_xyz.shape
    n = xyz_t.shape[2]
    nwc = _NCHUNK // _WPW
    nw = n // _WPW

    nn = lax.broadcasted_iota(jnp.int32, (_NCHUNK, 2 * nwc), 0)
    cc = lax.broadcasted_iota(jnp.int32, (_NCHUNK, 2 * nwc), 1)
    wsel = jnp.where(cc < nwc, cc, cc - nwc)
    same = (nn // _WPW) == wsel
    pw = jnp.where(cc < nwc, (1 << (nn % _WPW)).astype(jnp.float32), 1.0)
    wmat = jnp.where(same, pw, 0.0).astype(jnp.bfloat16)
    rr = lax.broadcasted_iota(jnp.int32, (nw, nw), 0)
    c2 = lax.broadcasted_iota(jnp.int32, (nw, nw), 1)
    tri = (rr <= c2).astype(jnp.bfloat16)
    wi16k = (lax.broadcasted_iota(jnp.int32, (_PBLK, nw), 1)
             .astype(jnp.float32) * 16384.0)

    return pl.pallas_call(
        _bq_body,
        grid=(b, p // _PBLK),
        in_specs=[
            pl.BlockSpec((1, _PBLK, 3), lambda bi, ji: (bi, ji, 0)),
            pl.BlockSpec((1, 3, n), lambda bi, ji: (bi, 0, 0)),
            pl.BlockSpec((_NCHUNK, 2 * nwc), lambda bi, ji: (0, 0)),
            pl.BlockSpec((nw, nw), lambda bi, ji: (0, 0)),
            pl.BlockSpec((_PBLK, nw), lambda bi, ji: (0, 0)),
        ],
        out_specs=pl.BlockSpec((1, _S, _PBLK), lambda bi, ji: (bi, 0, ji)),
        out_shape=jax.ShapeDtypeStruct((b, _S, p), jnp.int32),
    )(new_xyz, xyz_t, wmat, tri, wi16k)


def _sc_gather(idxt, feats, xyzf, nxpf, b_sz, c_sz, n, p):
    ntb = _NTILES // b_sz
    pchunk = p // ntb
    nvp = pchunk // 16
    out_ch = c_sz + 3

    mesh = plsc.VectorSubcoreMesh(core_axis_name="c", subcore_axis_name="s")

    def body(idx_hbm, feats_hbm, xyzf_hbm, nxp_hbm, out_hbm,
             idx_v, gi_v, mz_v, xyzrow_v, nxp_v,
             row0, row1, row2, row3, oxyz, o0, o1, o2, o3, dump_sh,
             rs0, rs1, rs2, rs3, os0, os1, os2, os3):
        rows = [row0, row1, row2, row3]
        outs = [o0, o1, o2, o3]
        rowsems = [rs0, rs1, rs2, rs3]
        outsems = [os0, os1, os2, os3]
        cid = lax.axis_index("c")
        sid = lax.axis_index("s")
        wid = sid * 2 + cid
        bb = wid // ntb
        pc = wid % ntb
        p0 = pc * pchunk
        pltpu.sync_copy(idx_hbm.at[bb, :, pl.ds(p0, pchunk)], idx_v)
        pltpu.sync_copy(xyzf_hbm.at[pl.ds(bb * 3 * n, 3 * n)],
                        xyzrow_v.at[pl.ds(0, 3 * n)])
        xyzrow_v[pl.ds(3 * n, 16)] = jnp.zeros((16,), jnp.float32)
        for d in range(3):
            pltpu.sync_copy(
                nxp_hbm.at[pl.ds((bb * 3 + d) * p + p0, pchunk)],
                nxp_v.at[pl.ds(d * pchunk, pchunk)])

        for r in range(_RING):
            pltpu.async_copy(
                feats_hbm.at[pl.ds((bb * c_sz + r) * n, n)],
                rows[r].at[pl.ds(0, n)], rowsems[r])
            rows[r][pl.ds(n, 16)] = jnp.zeros((16,), jnp.float32)
            pltpu.async_copy(outs[r], dump_sh, outsems[r])

        @plsc.parallel_loop(0, _S * nvp, 1, unroll=4)
        def _(v):
            s = v >> 4
            off = (v & (nvp - 1)) * 16
            iv = idx_v[s, pl.ds(off, 16)]
            valid = iv > 0
            gi_v[s, pl.ds(off, 16)] = jnp.where(valid, iv - 1, n)
            mz_v[s, pl.ds(off, 16)] = jnp.where(valid, 1.0, 0.0)

        for d in range(3):
            @plsc.parallel_loop(0, _S * nvp, 1, unroll=4)
            def _(v, d=d):
                s = v >> 4
                off = (v & (nvp - 1)) * 16
                g = gi_v[s, pl.ds(off, 16)]
                vals = plsc.load_gather(xyzrow_v, [g * 3 + d])
                nx = nxp_v[pl.ds(d * pchunk + off, 16)]
                m = mz_v[s, pl.ds(off, 16)]
                oxyz[s, pl.ds(off, 16)] = (vals - nx) * m
            pltpu.sync_copy(oxyz, out_hbm.at[bb, d, :, pl.ds(p0, pchunk)])

        def chan_group(k, carry):
            for r in range(_RING):
                c = k * _RING + r
                pltpu.make_async_copy(
                    feats_hbm.at[pl.ds((bb * c_sz) * n, n)],
                    rows[r].at[pl.ds(0, n)], rowsems[r]).wait()
                pltpu.make_async_copy(outs[r], dump_sh, outsems[r]).wait()

                @plsc.parallel_loop(0, _S * nvp, 1, unroll=8)
                def _(v, r=r):
                    s = v >> 4
                    off = (v & (nvp - 1)) * 16
                    g = gi_v[s, pl.ds(off, 16)]
                    vals = plsc.load_gather(rows[r], [g])
                    outs[r][s, pl.ds(off, 16)] = vals

                pltpu.async_copy(
                    outs[r],
                    out_hbm.at[bb, 3 + c, :, pl.ds(p0, pchunk)],
                    outsems[r])
                cnext = jnp.minimum(c + _RING, c_sz - 1)
                pltpu.async_copy(
                    feats_hbm.at[pl.ds((bb * c_sz + cnext) * n, n)],
                    rows[r].at[pl.ds(0, n)], rowsems[r])
            return carry
        lax.fori_loop(0, c_sz // _RING, chan_group, 0)

        for r in range(_RING):
            pltpu.make_async_copy(
                feats_hbm.at[pl.ds((bb * c_sz) * n, n)],
                rows[r].at[pl.ds(0, n)], rowsems[r]).wait()
            pltpu.make_async_copy(outs[r], dump_sh, outsems[r]).wait()

    k = pl.kernel(
        body,
        out_type=jax.ShapeDtypeStruct((b_sz, out_ch, _S, p), jnp.float32),
        mesh=mesh,
        compiler_params=pltpu.CompilerParams(needs_layout_passes=False),
        scratch_types=(
            [
                pltpu.VMEM((_S, pchunk), jnp.int32),
                pltpu.VMEM((_S, pchunk), jnp.int32),
                pltpu.VMEM((_S, pchunk), jnp.float32),
                pltpu.VMEM((3 * n + 16,), jnp.float32),
                pltpu.VMEM((3 * pchunk,), jnp.float32),
            ]
            + [pltpu.VMEM((n + 16,), jnp.float32)] * _RING
            + [pltpu.VMEM((_S, pchunk), jnp.float32)] * (1 + _RING)
            + [pltpu.VMEM_SHARED((_S, pchunk), jnp.float32)]
            + [pltpu.SemaphoreType.DMA] * (2 * _RING)
        ),
    )
    return k(idxt, feats, xyzf, nxpf)


def kernel(xyz, new_xyz, features):
    b, c, n = features.shape
    p = new_xyz.shape[1]
    xyz_t = jnp.transpose(xyz, (0, 2, 1))
    idxt = _ball_query(new_xyz, xyz_t)
    xyzf = xyz.reshape(b * 3 * n)
    nxpf = jnp.transpose(new_xyz, (0, 2, 1)).reshape(b * 3 * p)
    out = _sc_gather(idxt, features.reshape(b * c * n), xyzf, nxpf,
                     b, c, n, p)
    return jnp.transpose(out, (0, 1, 3, 2))


# speedup vs baseline: 1.2620x; 1.0027x over previous
import functools

import jax
import jax.numpy as jnp
from jax import lax
from jax.experimental import pallas as pl
from jax.experimental.pallas import tpu as pltpu
from jax.experimental.pallas import tpu_sc as plsc

_RADIUS2 = 0.25
_S = 32
_PBLK = 256
_NCHUNK = 2048
_WPW = 16
_NTILES = 32
_RING = 4


def _bq_body(new_ref, xyzt_ref, idx_ref):
    newb = new_ref[0]
    xyzt = xyzt_ref[0]
    n_pts = xyzt.shape[1]
    nchunks = n_pts // _NCHUNK
    nwc = _NCHUNK // _WPW
    nw = n_pts // _WPW

    sq_c = jnp.sum(newb * newb, axis=1, keepdims=True)

    nn = lax.broadcasted_iota(jnp.int32, (_NCHUNK, 2 * nwc), 0)
    cc = lax.broadcasted_iota(jnp.int32, (_NCHUNK, 2 * nwc), 1)
    wsel = jnp.where(cc < nwc, cc, cc - nwc)
    same = (nn // _WPW) == wsel
    pw = jnp.where(cc < nwc, (1 << (nn % _WPW)).astype(jnp.float32), 1.0)
    wmat = jnp.where(same, pw, 0.0).astype(jnp.bfloat16)

    words_cols = []
    cnts_cols = []
    for ci in range(nchunks):
        xc = lax.slice(xyzt, (0, ci * _NCHUNK), (3, (ci + 1) * _NCHUNK))
        cross = jnp.dot(newb, xc, preferred_element_type=jnp.float32)
        sq_x = jnp.sum(xc * xc, axis=0, keepdims=True)
        dist2 = sq_c + sq_x - 2.0 * cross
        m = (dist2 < _RADIUS2).astype(jnp.bfloat16)
        wc = jnp.dot(m, wmat, preferred_element_type=jnp.float32)
        words_cols.append(lax.slice(wc, (0, 0), (_PBLK, nwc)))
        cnts_cols.append(lax.slice(wc, (0, nwc), (_PBLK, 2 * nwc)))
    words = jnp.concatenate(words_cols, axis=1)
    cnts = jnp.concatenate(cnts_cols, axis=1)

    rr = lax.broadcasted_iota(jnp.int32, (nw, nw), 0)
    c2 = lax.broadcasted_iota(jnp.int32, (nw, nw), 1)
    tri = (rr <= c2).astype(jnp.bfloat16)
    cum = jnp.dot(cnts.astype(jnp.bfloat16), tri,
                  preferred_element_type=jnp.float32)
    prev = cum - cnts
    total = lax.slice(cum, (0, nw - 1), (_PBLK, nw))

    wi = lax.broadcasted_iota(jnp.int32, (_PBLK, nw), 1).astype(jnp.float32)
    combo_base = wi * 16384.0 + prev
    combo_cols = []
    word_cols = []
    for s in range(_S):
        sf = float(s)
        mf = jnp.logical_and(prev <= sf, cum > sf).astype(jnp.float32)
        combo_cols.append(jnp.sum(mf * combo_base, axis=1, keepdims=True))
        word_cols.append(jnp.sum(mf * words, axis=1, keepdims=True))
    combo = jnp.concatenate(combo_cols, axis=1)
    wordv = jnp.concatenate(word_cols, axis=1)

    cwi = combo.astype(jnp.int32)
    w_i = cwi >> 14
    base = cwi & 16383
    wv = wordv.astype(jnp.int32)
    si = lax.broadcasted_iota(jnp.int32, (_PBLK, _S), 1)
    j = si - base
    run = jnp.zeros((_PBLK, _S), jnp.int32)
    pos = jnp.zeros((_PBLK, _S), jnp.int32)
    for k in range(_WPW):
        bit = (wv >> k) & 1
        if k > 0:
            hit = jnp.logical_and(bit > 0, run == j)
            pos = pos + jnp.where(hit, k, 0)
        run = run + bit
    n_idx = w_i * _WPW + pos
    valid = si < total.astype(jnp.int32)
    idx = jnp.where(valid, n_idx + 1, 0)
    idx_ref[0] = jnp.transpose(idx, (1, 0))


def _ball_query(new_xyz, xyz_t):
    b, p, _ = new_xyz.shape
    n = xyz_t.shape[2]
    return pl.pallas_call(
        _bq_body,
        grid=(b, p // _PBLK),
        in_specs=[
            pl.BlockSpec((1, _PBLK, 3), lambda bi, ji: (bi, ji, 0)),
            pl.BlockSpec((1, 3, n), lambda bi, ji: (bi, 0, 0)),
        ],
        out_specs=pl.BlockSpec((1, _S, _PBLK), lambda bi, ji: (bi, 0, ji)),
        out_shape=jax.ShapeDtypeStruct((b, _S, p), jnp.int32),
    )(new_xyz, xyz_t)


def _sc_gather(idxt, feats, xyzf, nxpf, b_sz, c_sz, n, p):
    ntb = _NTILES // b_sz
    pchunk = p // ntb
    nvp = pchunk // 16
    out_ch = c_sz + 3

    mesh = plsc.VectorSubcoreMesh(core_axis_name="c", subcore_axis_name="s")

    def body(idx_hbm, feats_hbm, xyzf_hbm, nxp_hbm, out_hbm,
             idx_v, gi_v, mz_v, xyzrow_v, nxp_v,
             row0, row1, row2, row3, oxyz, o0, o1, o2, o3, dump_sh,
             rs0, rs1, rs2, rs3, os0, os1, os2, os3):
        rows = [row0, row1, row2, row3]
        outs = [o0, o1, o2, o3]
        rowsems = [rs0, rs1, rs2, rs3]
        outsems = [os0, os1, os2, os3]
        cid = lax.axis_index("c")
        sid = lax.axis_index("s")
        wid = sid * 2 + cid
        bb = wid // ntb
        pc = wid % ntb
        p0 = pc * pchunk
        pltpu.sync_copy(idx_hbm.at[bb, :, pl.ds(p0, pchunk)], idx_v)
        pltpu.sync_copy(xyzf_hbm.at[pl.ds(bb * 3 * n, 3 * n)],
                        xyzrow_v.at[pl.ds(0, 3 * n)])
        xyzrow_v[pl.ds(3 * n, 16)] = jnp.zeros((16,), jnp.float32)
        for d in range(3):
            pltpu.sync_copy(
                nxp_hbm.at[pl.ds((bb * 3 + d) * p + p0, pchunk)],
                nxp_v.at[pl.ds(d * pchunk, pchunk)])

        @plsc.parallel_loop(0, _S * nvp, 1, unroll=4)
        def _(v):
            s = v >> 4
            off = (v & (nvp - 1)) * 16
            iv = idx_v[s, pl.ds(off, 16)]
            valid = iv > 0
            gi_v[s, pl.ds(off, 16)] = jnp.where(valid, iv - 1, n)
            mz_v[s, pl.ds(off, 16)] = jnp.where(valid, 1.0, 0.0)

        for d in range(3):
            @plsc.parallel_loop(0, _S * nvp, 1, unroll=4)
            def _(v, d=d):
                s = v >> 4
                off = (v & (nvp - 1)) * 16
                g = gi_v[s, pl.ds(off, 16)]
                vals = plsc.load_gather(xyzrow_v, [g * 3 + d])
                nx = nxp_v[pl.ds(d * pchunk + off, 16)]
                m = mz_v[s, pl.ds(off, 16)]
                oxyz[s, pl.ds(off, 16)] = (vals - nx) * m
            pltpu.sync_copy(oxyz, out_hbm.at[bb, d, :, pl.ds(p0, pchunk)])

        for r in range(_RING):
            pltpu.async_copy(
                feats_hbm.at[pl.ds((bb * c_sz + r) * n, n)],
                rows[r].at[pl.ds(0, n)], rowsems[r])
            rows[r][pl.ds(n, 16)] = jnp.zeros((16,), jnp.float32)
            pltpu.async_copy(outs[r], dump_sh, outsems[r])

        def chan_group(k, carry):
            for r in range(_RING):
                c = k * _RING + r
                pltpu.make_async_copy(
                    feats_hbm.at[pl.ds((bb * c_sz) * n, n)],
                    rows[r].at[pl.ds(0, n)], rowsems[r]).wait()
                pltpu.make_async_copy(outs[r], dump_sh, outsems[r]).wait()

                @plsc.parallel_loop(0, _S * nvp, 1, unroll=8)
                def _(v, r=r):
                    s = v >> 4
                    off = (v & (nvp - 1)) * 16
                    g = gi_v[s, pl.ds(off, 16)]
                    vals = plsc.load_gather(rows[r], [g])
                    outs[r][s, pl.ds(off, 16)] = vals

                pltpu.async_copy(
                    outs[r],
                    out_hbm.at[bb, 3 + c, :, pl.ds(p0, pchunk)],
                    outsems[r])
                cnext = jnp.minimum(c + _RING, c_sz - 1)
                pltpu.async_copy(
                    feats_hbm.at[pl.ds((bb * c_sz + cnext) * n, n)],
                    rows[r].at[pl.ds(0, n)], rowsems[r])
            return carry
        lax.fori_loop(0, c_sz // _RING, chan_group, 0)

        for r in range(_RING):
            pltpu.make_async_copy(
                feats_hbm.at[pl.ds((bb * c_sz) * n, n)],
                rows[r].at[pl.ds(0, n)], rowsems[r]).wait()
            pltpu.make_async_copy(outs[r], dump_sh, outsems[r]).wait()

    k = pl.kernel(
        body,
        out_type=jax.ShapeDtypeStruct((b_sz, out_ch, _S, p), jnp.float32),
        mesh=mesh,
        compiler_params=pltpu.CompilerParams(needs_layout_passes=False),
        scratch_types=(
            [
                pltpu.VMEM((_S, pchunk), jnp.int32),
                pltpu.VMEM((_S, pchunk), jnp.int32),
                pltpu.VMEM((_S, pchunk), jnp.float32),
                pltpu.VMEM((3 * n + 16,), jnp.float32),
                pltpu.VMEM((3 * pchunk,), jnp.float32),
            ]
            + [pltpu.VMEM((n + 16,), jnp.float32)] * _RING
            + [pltpu.VMEM((_S, pchunk), jnp.float32)] * (1 + _RING)
            + [pltpu.VMEM_SHARED((_S, pchunk), jnp.float32)]
            + [pltpu.SemaphoreType.DMA] * (2 * _RING)
        ),
    )
    return k(idxt, feats, xyzf, nxpf)


def kernel(xyz, new_xyz, features):
    b, c, n = features.shape
    p = new_xyz.shape[1]
    xyz_t = jnp.transpose(xyz, (0, 2, 1))
    idxt = _ball_query(new_xyz, xyz_t)
    xyzf = xyz.reshape(b * 3 * n)
    nxpf = jnp.transpose(new_xyz, (0, 2, 1)).reshape(b * 3 * p)
    out = _sc_gather(idxt, features.reshape(b * c * n), xyzf, nxpf,
                     b, c, n, p)
    return jnp.transpose(out, (0, 1, 3, 2))
